# Initial kernel scaffold; baseline (speedup 1.0000x reference)
import jax
import jax.numpy as jnp
from jax.experimental import pallas as pl
from jax.experimental.pallas import tpu as pltpu
from jax.experimental.pallas import tpu_sc as plsc

_WINDOW = 256


def kernel(texts_tokenized, table):
    batch, seq = texts_tokenized.shape
    _, depth = table.shape
    num_idx = batch * seq
    ids = texts_tokenized.reshape(1, num_idx)

    mesh = plsc.VectorSubcoreMesh(core_axis_name="core",
                                  subcore_axis_name="subcore")

    @pl.kernel(out_type=jax.ShapeDtypeStruct((num_idx, depth), table.dtype),
               mesh=mesh)
    def gather_kernel(tab_hbm, i_hbm, o_hbm):
        def body(i_vmem, o_vmem):
            pltpu.sync_copy(tab_hbm.at[i_vmem.at[0]], o_vmem)

        pltpu.emit_pipeline(
            body,
            grid=(num_idx // _WINDOW,),
            in_specs=[pl.BlockSpec((1, _WINDOW), index_map=lambda i: (0, i))],
            out_specs=[pl.BlockSpec((_WINDOW, depth),
                                    index_map=lambda i: (i, 0))],
            core_axis_name=("core", "subcore"),
            dimension_semantics=(pltpu.PARALLEL,),
        )(i_hbm, o_hbm)

    out = gather_kernel(table, ids)
    return out.reshape(batch, seq, depth)


# speedup vs baseline: 2.6221x; 2.6221x over previous
import functools

import jax
from jax import lax
import jax.numpy as jnp
from jax.experimental import pallas as pl
from jax.experimental.pallas import tpu as pltpu
from jax.experimental.pallas import tpu_sc as plsc

_NC, _NS = 2, 16
_NW = _NC * _NS
_CHUNK = 320


def kernel(texts_tokenized, table):
    batch, seq = texts_tokenized.shape
    vocab, depth = table.shape
    num_idx = batch * seq
    b_per_w = num_idx // _NW
    n_chunks = b_per_w // _CHUNK
    assert b_per_w % _CHUNK == 0

    ids = texts_tokenized.reshape(num_idx)
    parity = ids & 1
    pair = ids >> 1
    idx_even = jnp.where(parity == 0, pair, -1)
    idx_odd = jnp.where(parity == 1, pair, -1)

    tab_a = table.reshape(vocab // 2, 2 * depth)
    flat = table.reshape(vocab * depth)
    tab_b = jnp.pad(flat[depth:], (0, depth)).reshape(vocab // 2, 2 * depth)

    mesh = plsc.VectorSubcoreMesh(core_axis_name="c", subcore_axis_name="s")

    @functools.partial(
        pl.kernel,
        mesh=mesh,
        out_type=jax.ShapeDtypeStruct((num_idx, depth), table.dtype),
        scratch_types=[
            pltpu.VMEM((_CHUNK,), jnp.int32),
            pltpu.VMEM((_CHUNK,), jnp.int32),
            pltpu.VMEM((_CHUNK, 2 * depth), table.dtype),
            pltpu.VMEM((_CHUNK, depth), table.dtype),
            pltpu.SemaphoreType.DMA,
        ],
    )
    def gather_kernel(ta_hbm, tb_hbm, ie_hbm, io_hbm, out_hbm,
                      ie_v, io_v, rows_v, out_v, sem):
        wid = lax.axis_index("s") * _NC + lax.axis_index("c")
        base = wid * b_per_w

        @pl.loop(0, n_chunks)
        def _(ci):
            o = base + ci * _CHUNK
            pltpu.sync_copy(ie_hbm.at[pl.ds(o, _CHUNK)], ie_v)
            pltpu.sync_copy(io_hbm.at[pl.ds(o, _CHUNK)], io_v)
            cp_e = pltpu.async_copy(
                ta_hbm.at[plsc.Indices(ie_v, ignored_value=-1)], rows_v, sem)
            cp_o = pltpu.async_copy(
                tb_hbm.at[plsc.Indices(io_v, ignored_value=-1)], rows_v, sem)
            cp_e.wait()
            cp_o.wait()
            @pl.loop(0, _CHUNK)
            def _(i):
                for q in range(depth // 16):
                    out_v[i, pl.ds(q * 16, 16)] = rows_v[i, pl.ds(q * 16, 16)]

            pltpu.sync_copy(out_v, out_hbm.at[pl.ds(o, _CHUNK)])

    out = gather_kernel(tab_a, tab_b, idx_even, idx_odd)
    return out.reshape(batch, seq, depth)


# speedup vs baseline: 2.9127x; 1.1108x over previous
import functools

import jax
from jax import lax
import jax.numpy as jnp
from jax.experimental import pallas as pl
from jax.experimental.pallas import tpu as pltpu
from jax.experimental.pallas import tpu_sc as plsc

_NC, _NS = 2, 16
_NW = _NC * _NS
_CHUNK = 200


def kernel(texts_tokenized, table):
    batch, seq = texts_tokenized.shape
    vocab, depth = table.shape
    num_idx = batch * seq
    b_per_w = num_idx // _NW
    n_chunks = b_per_w // _CHUNK
    assert b_per_w % _CHUNK == 0 and n_chunks % 2 == 0

    ids = texts_tokenized.reshape(num_idx)
    parity = ids & 1
    pair = ids >> 1
    idx_even = jnp.where(parity == 0, pair, -1).reshape(_NW * n_chunks, _CHUNK)
    idx_odd = jnp.where(parity == 1, pair, -1).reshape(_NW * n_chunks, _CHUNK)

    tab_a = table.reshape(vocab // 2, 2 * depth)
    flat = table.reshape(vocab * depth)
    tab_b = jnp.pad(flat[depth:], (0, depth)).reshape(vocab // 2, 2 * depth)

    mesh = plsc.VectorSubcoreMesh(core_axis_name="c", subcore_axis_name="s")

    @functools.partial(
        pl.kernel,
        mesh=mesh,
        out_type=jax.ShapeDtypeStruct((num_idx, depth), table.dtype),
        scratch_types=[
            pltpu.VMEM((_CHUNK,), jnp.int32),
            pltpu.VMEM((_CHUNK,), jnp.int32),
            pltpu.VMEM((_CHUNK,), jnp.int32),
            pltpu.VMEM((_CHUNK,), jnp.int32),
            pltpu.VMEM((_CHUNK, 2 * depth), table.dtype),
            pltpu.VMEM((_CHUNK, 2 * depth), table.dtype),
            pltpu.VMEM((_CHUNK, depth), table.dtype),
            pltpu.VMEM((_CHUNK, depth), table.dtype),
            pltpu.SemaphoreType.DMA,
            pltpu.SemaphoreType.DMA,
            pltpu.SemaphoreType.DMA,
            pltpu.SemaphoreType.DMA,
            pltpu.SemaphoreType.DMA,
            pltpu.SemaphoreType.DMA,
        ],
    )
    def gather_kernel(ta_hbm, tb_hbm, pe_hbm, po_hbm, out_hbm,
                      idxE0, idxO0, idxE1, idxO1, rows0, rows1, out0, out1,
                      semI0, semI1, semG0, semG1, semO0, semO1):
        wid = lax.axis_index("s") * _NC + lax.axis_index("c")
        base = wid * b_per_w
        cbase = wid * n_chunks

        def start_idx(ci, idxE, idxO, sem):
            pltpu.async_copy(pe_hbm.at[cbase + ci], idxE, sem)
            pltpu.async_copy(po_hbm.at[cbase + ci], idxO, sem)

        def wait_idx(idxE, idxO, sem):
            pltpu.make_async_copy(pe_hbm.at[cbase], idxE, sem).wait()
            pltpu.make_async_copy(po_hbm.at[cbase], idxO, sem).wait()

        def start_gather(idxE, idxO, rows_v, sem):
            pltpu.async_copy(
                ta_hbm.at[plsc.Indices(idxE, ignored_value=-1)],
                rows_v, sem)
            pltpu.async_copy(
                tb_hbm.at[plsc.Indices(idxO, ignored_value=-1)],
                rows_v, sem)

        def wait_gather(idxE, idxO, rows_v, sem):
            pltpu.make_async_copy(
                ta_hbm.at[plsc.Indices(idxE, ignored_value=-1)],
                rows_v, sem).wait()
            pltpu.make_async_copy(
                tb_hbm.at[plsc.Indices(idxO, ignored_value=-1)],
                rows_v, sem).wait()

        def compact(rows_v, out_v):
            @pl.loop(0, _CHUNK)
            def _(i):
                for q in range(depth // 16):
                    out_v[i, pl.ds(q * 16, 16)] = rows_v[i, pl.ds(q * 16, 16)]

        def start_out(ci, out_v, sem):
            pltpu.async_copy(out_v, out_hbm.at[pl.ds(base + ci * _CHUNK,
                                                     _CHUNK)], sem)

        def wait_out(out_v, sem):
            pltpu.make_async_copy(out_v, out_hbm.at[pl.ds(base, _CHUNK)],
                                  sem).wait()

        start_idx(0, idxE0, idxO0, semI0)
        wait_idx(idxE0, idxO0, semI0)
        start_gather(idxE0, idxO0, rows0, semG0)
        start_idx(1, idxE1, idxO1, semI1)

        @pl.loop(0, n_chunks // 2)
        def _(gi):
            g = gi * 2

            wait_gather(idxE0, idxO0, rows0, semG0)
            wait_idx(idxE1, idxO1, semI1)
            start_gather(idxE1, idxO1, rows1, semG1)

            @pl.when(g + 2 < n_chunks)
            def _():
                start_idx(g + 2, idxE0, idxO0, semI0)

            @pl.when(g >= 2)
            def _():
                wait_out(out0, semO0)

            compact(rows0, out0)
            start_out(g, out0, semO0)

            wait_gather(idxE1, idxO1, rows1, semG1)

            @pl.when(g + 2 < n_chunks)
            def _():
                wait_idx(idxE0, idxO0, semI0)
                start_gather(idxE0, idxO0, rows0, semG0)

            @pl.when(g + 3 < n_chunks)
            def _():
                start_idx(g + 3, idxE1, idxO1, semI1)

            @pl.when(g >= 2)
            def _():
                wait_out(out1, semO1)

            compact(rows1, out1)
            start_out(g + 1, out1, semO1)

        wait_out(out0, semO0)
        wait_out(out1, semO1)

    out = gather_kernel(tab_a, tab_b, idx_even, idx_odd)
    return out.reshape(batch, seq, depth)


# speedup vs baseline: 3.1760x; 1.0904x over previous
import functools

import jax
from jax import lax
import jax.numpy as jnp
from jax.experimental import pallas as pl
from jax.experimental.pallas import tpu as pltpu
from jax.experimental.pallas import tpu_sc as plsc

_NC, _NS = 2, 16
_NW = _NC * _NS
_CHUNK = 160
_ROLL_BLK = 5000


def _roll_body(x_ref, o_ref):
    half = x_ref.shape[1] // 2
    o_ref[:, :half] = x_ref[:, half:]
    o_ref[:, half:] = x_ref[:, :half]


def kernel(texts_tokenized, table):
    batch, seq = texts_tokenized.shape
    vocab, depth = table.shape
    num_idx = batch * seq
    b_per_w = num_idx // _NW
    n_chunks = b_per_w // _CHUNK
    assert b_per_w % _CHUNK == 0 and n_chunks % 2 == 0
    assert _CHUNK % 16 == 0

    ids = texts_tokenized.reshape(num_idx)
    tab_a = table.reshape(vocab // 2, 2 * depth)
    tab_b = pl.pallas_call(
        _roll_body,
        out_shape=jax.ShapeDtypeStruct((vocab // 2, 2 * depth), table.dtype),
        grid=(vocab // 2 // _ROLL_BLK,),
        in_specs=[pl.BlockSpec((_ROLL_BLK, 2 * depth), lambda i: (i, 0))],
        out_specs=pl.BlockSpec((_ROLL_BLK, 2 * depth), lambda i: (i, 0)),
    )(tab_a)

    mesh = plsc.VectorSubcoreMesh(core_axis_name="c", subcore_axis_name="s")

    @functools.partial(
        pl.kernel,
        mesh=mesh,
        out_type=jax.ShapeDtypeStruct((num_idx, depth), table.dtype),
        scratch_types=[
            pltpu.VMEM((_CHUNK,), jnp.int32),
            pltpu.VMEM((_CHUNK,), jnp.int32),
            pltpu.VMEM((_CHUNK,), jnp.int32),
            pltpu.VMEM((_CHUNK,), jnp.int32),
            pltpu.VMEM((_CHUNK,), jnp.int32),
            pltpu.VMEM((_CHUNK,), jnp.int32),
            pltpu.VMEM((_CHUNK, 2 * depth), table.dtype),
            pltpu.VMEM((_CHUNK, 2 * depth), table.dtype),
            pltpu.VMEM((_CHUNK, depth), table.dtype),
            pltpu.VMEM((_CHUNK, depth), table.dtype),
            pltpu.SemaphoreType.DMA,
            pltpu.SemaphoreType.DMA,
            pltpu.SemaphoreType.DMA,
            pltpu.SemaphoreType.DMA,
            pltpu.SemaphoreType.DMA,
            pltpu.SemaphoreType.DMA,
        ],
    )
    def gather_kernel(ta_hbm, tb_hbm, ids_hbm, out_hbm,
                      ids0, ids1, idxE0, idxO0, idxE1, idxO1,
                      rows0, rows1, out0, out1,
                      semI0, semI1, semG0, semG1, semO0, semO1):
        wid = lax.axis_index("s") * _NC + lax.axis_index("c")
        base = wid * b_per_w

        def start_idx(ci, ids_v, sem):
            pltpu.async_copy(ids_hbm.at[pl.ds(base + ci * _CHUNK, _CHUNK)],
                             ids_v, sem)

        def wait_idx(ids_v, sem):
            pltpu.make_async_copy(ids_hbm.at[pl.ds(base, _CHUNK)],
                                  ids_v, sem).wait()

        def compute_idx(ids_v, idxE, idxO):
            neg1 = jnp.full((16,), -1, jnp.int32)

            @pl.loop(0, _CHUNK // 16)
            def _(g):
                v = ids_v[pl.ds(g * 16, 16)]
                p = jnp.right_shift(v, 1)
                even = (v & 1) == 0
                idxE[pl.ds(g * 16, 16)] = jnp.where(even, p, neg1)
                idxO[pl.ds(g * 16, 16)] = jnp.where(even, neg1, p)

        def start_gather(idxE, idxO, rows_v, sem):
            pltpu.async_copy(
                ta_hbm.at[plsc.Indices(idxE, ignored_value=-1)], rows_v, sem)
            pltpu.async_copy(
                tb_hbm.at[plsc.Indices(idxO, ignored_value=-1)], rows_v, sem)

        def wait_gather(idxE, idxO, rows_v, sem):
            pltpu.make_async_copy(
                ta_hbm.at[plsc.Indices(idxE, ignored_value=-1)],
                rows_v, sem).wait()
            pltpu.make_async_copy(
                tb_hbm.at[plsc.Indices(idxO, ignored_value=-1)],
                rows_v, sem).wait()

        def compact(rows_v, out_v):
            @pl.loop(0, _CHUNK)
            def _(i):
                for q in range(depth // 16):
                    out_v[i, pl.ds(q * 16, 16)] = rows_v[i, pl.ds(q * 16, 16)]

        def start_out(ci, out_v, sem):
            pltpu.async_copy(out_v, out_hbm.at[pl.ds(base + ci * _CHUNK,
                                                     _CHUNK)], sem)

        def wait_out(out_v, sem):
            pltpu.make_async_copy(out_v, out_hbm.at[pl.ds(base, _CHUNK)],
                                  sem).wait()

        start_idx(0, ids0, semI0)
        wait_idx(ids0, semI0)
        compute_idx(ids0, idxE0, idxO0)
        start_gather(idxE0, idxO0, rows0, semG0)
        start_idx(1, ids1, semI1)

        @pl.loop(0, n_chunks // 2)
        def _(gi):
            g = gi * 2

            wait_gather(idxE0, idxO0, rows0, semG0)
            wait_idx(ids1, semI1)
            compute_idx(ids1, idxE1, idxO1)
            start_gather(idxE1, idxO1, rows1, semG1)

            @pl.when(g + 2 < n_chunks)
            def _():
                start_idx(g + 2, ids0, semI0)

            @pl.when(g >= 2)
            def _():
                wait_out(out0, semO0)

            compact(rows0, out0)
            start_out(g, out0, semO0)

            wait_gather(idxE1, idxO1, rows1, semG1)

            @pl.when(g + 2 < n_chunks)
            def _():
                wait_idx(ids0, semI0)
                compute_idx(ids0, idxE0, idxO0)
                start_gather(idxE0, idxO0, rows0, semG0)

            @pl.when(g + 3 < n_chunks)
            def _():
                start_idx(g + 3, ids1, semI1)

            @pl.when(g >= 2)
            def _():
                wait_out(out1, semO1)

            compact(rows1, out1)
            start_out(g + 1, out1, semO1)

        wait_out(out0, semO0)
        wait_out(out1, semO1)

    out = gather_kernel(tab_a, tab_b, ids)
    return out.reshape(batch, seq, depth)


# speedup vs baseline: 3.1977x; 1.0068x over previous
import functools

import jax
from jax import lax
import jax.numpy as jnp
from jax.experimental import pallas as pl
from jax.experimental.pallas import tpu as pltpu
from jax.experimental.pallas import tpu_sc as plsc

_NC, _NS = 2, 16
_NW = _NC * _NS
_CHUNK = 160
_ROLL_BLK = 5000


def _roll_body(x_ref, o_ref):
    half = x_ref.shape[1] // 2
    o_ref[:, :half] = x_ref[:, half:]
    o_ref[:, half:] = x_ref[:, :half]


def kernel(texts_tokenized, table):
    batch, seq = texts_tokenized.shape
    vocab, depth = table.shape
    num_idx = batch * seq
    b_per_w = num_idx // _NW
    n_chunks = b_per_w // _CHUNK
    assert b_per_w % _CHUNK == 0 and n_chunks % 2 == 0
    assert _CHUNK % 16 == 0

    ids = texts_tokenized.reshape(num_idx)
    tab_a = table.reshape(vocab // 2, 2 * depth)
    tab_b = pl.pallas_call(
        _roll_body,
        out_shape=jax.ShapeDtypeStruct((vocab // 2, 2 * depth), table.dtype),
        grid=(vocab // 2 // _ROLL_BLK,),
        in_specs=[pl.BlockSpec((_ROLL_BLK, 2 * depth), lambda i: (i, 0))],
        out_specs=pl.BlockSpec((_ROLL_BLK, 2 * depth), lambda i: (i, 0)),
    )(tab_a)

    mesh = plsc.VectorSubcoreMesh(core_axis_name="c", subcore_axis_name="s")

    @functools.partial(
        pl.kernel,
        mesh=mesh,
        out_type=jax.ShapeDtypeStruct((num_idx, depth), table.dtype),
        scratch_types=[
            pltpu.VMEM((_CHUNK,), jnp.int32),
            pltpu.VMEM((_CHUNK,), jnp.int32),
            pltpu.VMEM((_CHUNK,), jnp.int32),
            pltpu.VMEM((_CHUNK,), jnp.int32),
            pltpu.VMEM((_CHUNK,), jnp.int32),
            pltpu.VMEM((_CHUNK,), jnp.int32),
            pltpu.VMEM((_CHUNK, 2 * depth), table.dtype),
            pltpu.VMEM((_CHUNK, 2 * depth), table.dtype),
            pltpu.VMEM((_CHUNK, depth), table.dtype),
            pltpu.VMEM((_CHUNK, depth), table.dtype),
            pltpu.SemaphoreType.DMA,
            pltpu.SemaphoreType.DMA,
            pltpu.SemaphoreType.DMA,
            pltpu.SemaphoreType.DMA,
            pltpu.SemaphoreType.DMA,
            pltpu.SemaphoreType.DMA,
        ],
    )
    def gather_kernel(ta_hbm, tb_hbm, ids_hbm, out_hbm,
                      ids0, ids1, idxE0, idxO0, idxE1, idxO1,
                      rows0, rows1, out0, out1,
                      semI0, semI1, semG0, semG1, semO0, semO1):
        wid = lax.axis_index("s") * _NC + lax.axis_index("c")
        base = wid * b_per_w

        def start_idx(ci, ids_v, sem):
            pltpu.async_copy(ids_hbm.at[pl.ds(base + ci * _CHUNK, _CHUNK)],
                             ids_v, sem)

        def wait_idx(ids_v, sem):
            pltpu.make_async_copy(ids_hbm.at[pl.ds(base, _CHUNK)],
                                  ids_v, sem).wait()

        def compute_idx(ids_v, idxE, idxO):
            neg1 = jnp.full((16,), -1, jnp.int32)

            @pl.loop(0, _CHUNK // 16)
            def _(g):
                v = ids_v[pl.ds(g * 16, 16)]
                p = jnp.right_shift(v, 1)
                even = (v & 1) == 0
                idxE[pl.ds(g * 16, 16)] = jnp.where(even, p, neg1)
                idxO[pl.ds(g * 16, 16)] = jnp.where(even, neg1, p)

        def start_gather(idxE, idxO, rows_v, sem):
            pltpu.async_copy(
                ta_hbm.at[plsc.Indices(idxE, ignored_value=-1)], rows_v, sem)
            pltpu.async_copy(
                tb_hbm.at[plsc.Indices(idxO, ignored_value=-1)], rows_v, sem)

        def wait_gather(idxE, idxO, rows_v, sem):
            pltpu.make_async_copy(
                ta_hbm.at[plsc.Indices(idxE, ignored_value=-1)],
                rows_v, sem).wait()
            pltpu.make_async_copy(
                tb_hbm.at[plsc.Indices(idxO, ignored_value=-1)],
                rows_v, sem).wait()

        def compact(rows_v, out_v):
            @functools.partial(plsc.parallel_loop, 0, _CHUNK, unroll=4)
            def _(i):
                for q in range(depth // 16):
                    out_v[i, pl.ds(q * 16, 16)] = rows_v[i, pl.ds(q * 16, 16)]

        def start_out(ci, out_v, sem):
            pltpu.async_copy(out_v, out_hbm.at[pl.ds(base + ci * _CHUNK,
                                                     _CHUNK)], sem)

        def wait_out(out_v, sem):
            pltpu.make_async_copy(out_v, out_hbm.at[pl.ds(base, _CHUNK)],
                                  sem).wait()

        start_idx(0, ids0, semI0)
        wait_idx(ids0, semI0)
        compute_idx(ids0, idxE0, idxO0)
        start_gather(idxE0, idxO0, rows0, semG0)
        start_idx(1, ids1, semI1)

        @pl.loop(0, n_chunks // 2)
        def _(gi):
            g = gi * 2

            wait_gather(idxE0, idxO0, rows0, semG0)
            wait_idx(ids1, semI1)
            compute_idx(ids1, idxE1, idxO1)
            start_gather(idxE1, idxO1, rows1, semG1)

            @pl.when(g + 2 < n_chunks)
            def _():
                start_idx(g + 2, ids0, semI0)

            @pl.when(g >= 2)
            def _():
                wait_out(out0, semO0)

            compact(rows0, out0)
            start_out(g, out0, semO0)

            wait_gather(idxE1, idxO1, rows1, semG1)

            @pl.when(g + 2 < n_chunks)
            def _():
                wait_idx(ids0, semI0)
                compute_idx(ids0, idxE0, idxO0)
                start_gather(idxE0, idxO0, rows0, semG0)

            @pl.when(g + 3 < n_chunks)
            def _():
                start_idx(g + 3, ids1, semI1)

            @pl.when(g >= 2)
            def _():
                wait_out(out1, semO1)

            compact(rows1, out1)
            start_out(g + 1, out1, semO1)

        wait_out(out0, semO0)
        wait_out(out1, semO1)

    out = gather_kernel(tab_a, tab_b, ids)
    return out.reshape(batch, seq, depth)


# speedup vs baseline: 3.5090x; 1.0973x over previous
import functools

import jax
from jax import lax
import jax.numpy as jnp
from jax.experimental import pallas as pl
from jax.experimental.pallas import tpu as pltpu
from jax.experimental.pallas import tpu_sc as plsc

_NC, _NS = 2, 16
_NW = _NC * _NS
_CHUNK = 160
_PAD_BLK = 5000


def _pad_body(x_ref, o_ref):
    depth = x_ref.shape[1]
    o_ref[:, :depth] = x_ref[...]
    o_ref[:, depth:] = jnp.zeros_like(x_ref)


def kernel(texts_tokenized, table):
    batch, seq = texts_tokenized.shape
    vocab, depth = table.shape
    num_idx = batch * seq
    b_per_w = num_idx // _NW
    n_chunks = b_per_w // _CHUNK
    assert b_per_w % _CHUNK == 0 and n_chunks % 2 == 0
    assert _CHUNK % 16 == 0

    ids = texts_tokenized.reshape(num_idx)
    tab_pad = pl.pallas_call(
        _pad_body,
        out_shape=jax.ShapeDtypeStruct((vocab, 2 * depth), table.dtype),
        grid=(vocab // _PAD_BLK,),
        in_specs=[pl.BlockSpec((_PAD_BLK, depth), lambda i: (i, 0))],
        out_specs=pl.BlockSpec((_PAD_BLK, 2 * depth), lambda i: (i, 0)),
    )(table)

    mesh = plsc.VectorSubcoreMesh(core_axis_name="c", subcore_axis_name="s")

    @functools.partial(
        pl.kernel,
        mesh=mesh,
        out_type=jax.ShapeDtypeStruct((num_idx, depth), table.dtype),
        scratch_types=[
            pltpu.VMEM((_CHUNK,), jnp.int32),
            pltpu.VMEM((_CHUNK,), jnp.int32),
            pltpu.VMEM((_CHUNK, 2 * depth), table.dtype),
            pltpu.VMEM((_CHUNK, 2 * depth), table.dtype),
            pltpu.VMEM((_CHUNK, depth), table.dtype),
            pltpu.VMEM((_CHUNK, depth), table.dtype),
            pltpu.SemaphoreType.DMA,
            pltpu.SemaphoreType.DMA,
            pltpu.SemaphoreType.DMA,
            pltpu.SemaphoreType.DMA,
            pltpu.SemaphoreType.DMA,
            pltpu.SemaphoreType.DMA,
        ],
    )
    def gather_kernel(tp_hbm, ids_hbm, out_hbm,
                      ids0, ids1, rows0, rows1, out0, out1,
                      semI0, semI1, semG0, semG1, semO0, semO1):
        wid = lax.axis_index("s") * _NC + lax.axis_index("c")
        base = wid * b_per_w

        def start_idx(ci, ids_v, sem):
            pltpu.async_copy(ids_hbm.at[pl.ds(base + ci * _CHUNK, _CHUNK)],
                             ids_v, sem)

        def wait_idx(ids_v, sem):
            pltpu.make_async_copy(ids_hbm.at[pl.ds(base, _CHUNK)],
                                  ids_v, sem).wait()

        def start_gather(ids_v, rows_v, sem):
            pltpu.async_copy(
                tp_hbm.at[plsc.Indices(ids_v, ignored_value=-1)], rows_v, sem)

        def wait_gather(ids_v, rows_v, sem):
            pltpu.make_async_copy(
                tp_hbm.at[plsc.Indices(ids_v, ignored_value=-1)],
                rows_v, sem).wait()

        def compact(rows_v, out_v):
            @functools.partial(plsc.parallel_loop, 0, _CHUNK, unroll=4)
            def _(i):
                for q in range(depth // 16):
                    out_v[i, pl.ds(q * 16, 16)] = rows_v[i, pl.ds(q * 16, 16)]

        def start_out(ci, out_v, sem):
            pltpu.async_copy(out_v, out_hbm.at[pl.ds(base + ci * _CHUNK,
                                                     _CHUNK)], sem)

        def wait_out(out_v, sem):
            pltpu.make_async_copy(out_v, out_hbm.at[pl.ds(base, _CHUNK)],
                                  sem).wait()

        start_idx(0, ids0, semI0)
        wait_idx(ids0, semI0)
        start_gather(ids0, rows0, semG0)
        start_idx(1, ids1, semI1)

        @pl.loop(0, n_chunks // 2)
        def _(gi):
            g = gi * 2

            wait_gather(ids0, rows0, semG0)
            wait_idx(ids1, semI1)
            start_gather(ids1, rows1, semG1)

            @pl.when(g + 2 < n_chunks)
            def _():
                start_idx(g + 2, ids0, semI0)

            @pl.when(g >= 2)
            def _():
                wait_out(out0, semO0)

            compact(rows0, out0)
            start_out(g, out0, semO0)

            wait_gather(ids1, rows1, semG1)

            @pl.when(g + 2 < n_chunks)
            def _():
                wait_idx(ids0, semI0)
                start_gather(ids0, rows0, semG0)

            @pl.when(g + 3 < n_chunks)
            def _():
                start_idx(g + 3, ids1, semI1)

            @pl.when(g >= 2)
            def _():
                wait_out(out1, semO1)

            compact(rows1, out1)
            start_out(g + 1, out1, semO1)

        wait_out(out0, semO0)
        wait_out(out1, semO1)

    out = gather_kernel(tab_pad, ids)
    return out.reshape(batch, seq, depth)
